# Initial kernel scaffold; baseline (speedup 1.0000x reference)
import jax
import jax.numpy as jnp
from jax.experimental import pallas as pl


def kernel(x, edge_index, W1, b1, gamma, beta, W2, b2):
    raise NotImplementedError("write your pallas kernel here")


# speedup vs baseline: 5.5826x; 5.5826x over previous
import functools

import jax
import jax.numpy as jnp
from jax import lax
from jax.experimental import pallas as pl
from jax.experimental.pallas import tpu as pltpu
from jax.experimental.pallas import tpu_sc as plsc

N = 10000
E = 320000
D = 128

NC = 2
NS = 16
NW = NC * NS

EPW = E // NW
CHUNK = 80
NCHUNK = EPW // CHUNK
NPAD = 10240
RPT = NPAD // NS
ZROWS = 128


def _sc_aggregate(x, src, dst):
    mesh = plsc.VectorSubcoreMesh(core_axis_name="c", subcore_axis_name="s")

    @functools.partial(
        pl.kernel,
        mesh=mesh,
        out_type=jax.ShapeDtypeStruct((NC, NPAD, D), jnp.float32),
        scratch_types=[
            pltpu.VMEM((CHUNK,), jnp.int32),
            pltpu.VMEM((CHUNK,), jnp.int32),
            pltpu.VMEM((CHUNK, D), jnp.float32),
            pltpu.VMEM((ZROWS, D), jnp.float32),
            pltpu.VMEM_SHARED((NPAD, D), jnp.float32),
        ],
    )
    def agg_kernel(x_hbm, src_hbm, dst_hbm, out_hbm,
                   sidx_v, didx_v, rows_v, zero_v, acc):
        c = lax.axis_index("c")
        s = lax.axis_index("s")
        wid = s * NC + c

        def zero_row(r, carry):
            for c0 in range(0, D, 16):
                zero_v[r, pl.ds(c0, 16)] = jnp.zeros((16,), jnp.float32)
            return carry
        lax.fori_loop(0, ZROWS, zero_row, 0)
        for t in range(RPT // ZROWS):
            pltpu.sync_copy(zero_v, acc.at[pl.ds(s * RPT + t * ZROWS, ZROWS)])
        plsc.subcore_barrier()

        ebase = wid * EPW

        def body(j, carry):
            off = ebase + j * CHUNK
            pltpu.sync_copy(src_hbm.at[pl.ds(off, CHUNK)], sidx_v)
            pltpu.sync_copy(dst_hbm.at[pl.ds(off, CHUNK)], didx_v)
            pltpu.sync_copy(x_hbm.at[sidx_v], rows_v)
            pltpu.sync_copy(rows_v, acc.at[didx_v], add=True)
            return carry
        lax.fori_loop(0, NCHUNK, body, 0)
        plsc.subcore_barrier()

        for t in range(RPT // ZROWS):
            r0 = s * RPT + t * ZROWS
            pltpu.sync_copy(acc.at[pl.ds(r0, ZROWS)],
                            out_hbm.at[c, pl.ds(r0, ZROWS)])

    return agg_kernel(x, src, dst)


def _mlp_kernel(x_ref, p_ref, w1_ref, b1_ref, g_ref, be_ref, w2_ref, b2_ref,
                o_ref):
    h = x_ref[...] + p_ref[0, :N, :] + p_ref[1, :N, :]
    z = lax.dot_general(h, w1_ref[...], (((1,), (1,)), ((), ())),
                        preferred_element_type=jnp.float32)
    z = jnp.maximum(z + b1_ref[...], 0.0)
    mean = jnp.mean(z, axis=0, keepdims=True)
    var = jnp.mean(z * z, axis=0, keepdims=True) - mean * mean
    scale = g_ref[...] * lax.rsqrt(var + 1e-5)
    zn = (z - mean) * scale + be_ref[...]
    o_ref[...] = lax.dot_general(zn, w2_ref[...], (((1,), (1,)), ((), ())),
                                 preferred_element_type=jnp.float32) + b2_ref[...]


def _mlp(x, parts, W1, b1, gamma, beta, W2, b2):
    return pl.pallas_call(
        _mlp_kernel,
        out_shape=jax.ShapeDtypeStruct((N, D), jnp.float32),
    )(x, parts, W1, b1.reshape(1, D), gamma.reshape(1, D),
      beta.reshape(1, D), W2, b2.reshape(1, D))


def kernel(x, edge_index, W1, b1, gamma, beta, W2, b2):
    src = edge_index[0]
    dst = edge_index[1]
    parts = _sc_aggregate(x, src, dst)
    return _mlp(x, parts, W1, b1, gamma, beta, W2, b2)
